# baseline (speedup 1.0000x reference)
import functools

import jax
import jax.numpy as jnp
from jax import lax
from jax.experimental import pallas as pl
from jax.experimental.pallas import tpu as pltpu
from jax.experimental.pallas import tpu_sc as plsc

E_DIM = 64
N_CODES = 1024
ROWS = 32 * 1024
BLK = 1024
N_BLK = ROWS // BLK


def _dist_argmin_body(x_ref, e_ref, idx_ref, perp_ref, counts_ref):
    i = pl.program_id(0)
    x = x_ref[...]
    e = e_ref[...]
    a_sq = jnp.sum(x * x, axis=1, keepdims=True)
    ab = lax.dot_general(x + x, e, (((1,), (1,)), ((), ())))
    b_sq = jnp.sum(e * e, axis=1)[None, :]
    dist = (a_sq - ab) + b_sq
    m = jnp.min(dist, axis=1, keepdims=True)
    eqm = dist == m
    lane_f = lax.broadcasted_iota(jnp.int32, (1, N_CODES), 1).astype(jnp.float32)
    idx_f = jnp.min(jnp.where(eqm, lane_f, float(N_CODES)), axis=1)
    idx_ref[...] = idx_f.astype(jnp.int32).reshape(BLK // 128, 128)
    cnt = jnp.sum(jnp.where(eqm, 1.0, 0.0), axis=0, keepdims=True)

    @pl.when(i == 0)
    def _():
        counts_ref[...] = cnt

    @pl.when(i > 0)
    def _():
        counts_ref[...] = counts_ref[...] + cnt

    @pl.when(i == N_BLK - 1)
    def _():
        p = counts_ref[...] * (1.0 / ROWS)
        ent = jnp.sum(p * jnp.log(p + 1e-10), axis=(0, 1), keepdims=True)
        perp_ref[...] = jnp.exp(-ent)


def _tc_stage(flat, emb):
    return pl.pallas_call(
        _dist_argmin_body,
        grid=(N_BLK,),
        in_specs=[
            pl.BlockSpec((BLK, E_DIM), lambda i: (i, 0)),
            pl.BlockSpec((N_CODES, E_DIM), lambda i: (0, 0)),
        ],
        out_specs=[
            pl.BlockSpec((BLK // 128, 128), lambda i: (i, 0)),
            pl.BlockSpec((1, 1), lambda i: (0, 0)),
        ],
        out_shape=[
            jax.ShapeDtypeStruct((ROWS // 128, 128), jnp.int32),
            jax.ShapeDtypeStruct((1, 1), jnp.float32),
        ],
        scratch_shapes=[pltpu.VMEM((1, N_CODES), jnp.float32)],
    )(flat, emb)


def _sc_gather(table_p, idx_flat):
    info = plsc.get_sparse_core_info()
    nc, ns = info.num_cores, info.num_subcores
    nw = nc * ns
    b_per_w = ROWS // nw
    mesh = plsc.VectorSubcoreMesh(core_axis_name="c", subcore_axis_name="s")

    @functools.partial(
        pl.kernel,
        mesh=mesh,
        compiler_params=pltpu.CompilerParams(use_tc_tiling_on_sc=False),
        out_type=jax.ShapeDtypeStruct((ROWS, 128), jnp.float32),
        scratch_types=[
            pltpu.VMEM((b_per_w // 2,), jnp.int32),
            pltpu.VMEM((b_per_w // 2, 128), jnp.float32),
            pltpu.SemaphoreType.DMA,
        ],
    )
    def gather_k(table_hbm, idx_hbm, out_hbm, idx_v, rows_v, sem):
        wid = lax.axis_index("s") * nc + lax.axis_index("c")
        half = b_per_w // 2
        for j in range(2):
            base = wid * b_per_w + j * half
            pltpu.sync_copy(idx_hbm.at[pl.ds(base, half)], idx_v)
            pltpu.async_copy(table_hbm.at[idx_v], rows_v, sem).wait()
            pltpu.sync_copy(rows_v, out_hbm.at[pl.ds(base, half)])

    return gather_k(table_p, idx_flat)


def _retile_body(q_ref, out_ref):
    out_ref[...] = q_ref[:, :E_DIM].reshape(1, BLK, E_DIM)


def _retile(q_pad, batch, seq):
    return pl.pallas_call(
        _retile_body,
        grid=(batch,),
        in_specs=[pl.BlockSpec((BLK, 128), lambda i: (i, 0))],
        out_specs=pl.BlockSpec((1, BLK, E_DIM), lambda i: (i, 0, 0)),
        out_shape=jax.ShapeDtypeStruct((batch, seq, E_DIM), jnp.float32),
    )(q_pad)


def kernel(inputs, embeddings):
    flat = inputs.reshape(ROWS, E_DIM)
    idx2d, perp = _tc_stage(flat, embeddings)
    table_p = jnp.pad(embeddings, ((0, 0), (0, 128 - E_DIM)))
    q_pad = _sc_gather(table_p, idx2d.reshape(ROWS))
    quant = _retile(q_pad, inputs.shape[0], inputs.shape[1])
    return (
        quant,
        idx2d.reshape(inputs.shape[:-1]),
        perp[0, 0],
    )


# speedup vs baseline: 1.2555x; 1.2555x over previous
import functools

import jax
import jax.numpy as jnp
from jax import lax
from jax.experimental import pallas as pl
from jax.experimental.pallas import tpu as pltpu
from jax.experimental.pallas import tpu_sc as plsc

E_DIM = 64
N_CODES = 1024
ROWS = 32 * 1024
BLK = 2048
N_BLK = ROWS // BLK


def _dist_argmin_body(x_ref, e_ref, idx_ref, perp_ref, counts_ref):
    i = pl.program_id(0)
    x = x_ref[...]
    e = e_ref[...]
    a_sq = jnp.sum(x * x, axis=1, keepdims=True)
    ab = lax.dot_general(x + x, e, (((1,), (1,)), ((), ())))
    b_sq = jnp.sum(e * e, axis=1)[None, :]
    dist = (a_sq - ab) + b_sq
    m = jnp.min(dist, axis=1, keepdims=True)
    eqm = dist == m
    lane_f = lax.broadcasted_iota(jnp.int32, (1, N_CODES), 1).astype(jnp.float32)
    idx_f = jnp.min(jnp.where(eqm, lane_f, float(N_CODES)), axis=1)
    idx_ref[...] = idx_f.astype(jnp.int32).reshape(BLK // 128, 128)
    cnt = jnp.sum(jnp.where(eqm, 1.0, 0.0), axis=0, keepdims=True)

    @pl.when(i == 0)
    def _():
        counts_ref[...] = cnt

    @pl.when(i > 0)
    def _():
        counts_ref[...] = counts_ref[...] + cnt

    @pl.when(i == N_BLK - 1)
    def _():
        p = counts_ref[...] * (1.0 / ROWS)
        ent = jnp.sum(p * jnp.log(p + 1e-10), axis=(0, 1), keepdims=True)
        perp_ref[...] = jnp.exp(-ent)


def _tc_stage(flat, emb):
    return pl.pallas_call(
        _dist_argmin_body,
        grid=(N_BLK,),
        in_specs=[
            pl.BlockSpec((BLK, E_DIM), lambda i: (i, 0)),
            pl.BlockSpec((N_CODES, E_DIM), lambda i: (0, 0)),
        ],
        out_specs=[
            pl.BlockSpec((BLK // 128, 128), lambda i: (i, 0)),
            pl.BlockSpec((1, 1), lambda i: (0, 0)),
        ],
        out_shape=[
            jax.ShapeDtypeStruct((ROWS // 128, 128), jnp.int32),
            jax.ShapeDtypeStruct((1, 1), jnp.float32),
        ],
        scratch_shapes=[pltpu.VMEM((1, N_CODES), jnp.float32)],
    )(flat, emb)


def _sc_gather(table, idx_flat):
    info = plsc.get_sparse_core_info()
    nc, ns = info.num_cores, info.num_subcores
    nw = nc * ns
    b_per_w = ROWS // nw
    mesh = plsc.VectorSubcoreMesh(core_axis_name="c", subcore_axis_name="s")

    @functools.partial(
        pl.kernel,
        mesh=mesh,
        compiler_params=pltpu.CompilerParams(use_tc_tiling_on_sc=False),
        out_type=jax.ShapeDtypeStruct((ROWS, E_DIM), jnp.float32),
        scratch_types=[
            pltpu.VMEM((b_per_w,), jnp.int32),
            pltpu.VMEM((b_per_w, E_DIM), jnp.float32),
            pltpu.SemaphoreType.DMA,
        ],
    )
    def gather_k(table_hbm, idx_hbm, out_hbm, idx_v, rows_v, sem):
        wid = lax.axis_index("s") * nc + lax.axis_index("c")
        base = wid * b_per_w
        pltpu.sync_copy(idx_hbm.at[pl.ds(base, b_per_w)], idx_v)
        pltpu.async_copy(table_hbm.at[idx_v], rows_v, sem).wait()
        pltpu.sync_copy(rows_v, out_hbm.at[pl.ds(base, b_per_w)])

    return gather_k(table, idx_flat)


def kernel(inputs, embeddings):
    flat = inputs.reshape(ROWS, E_DIM)
    idx2d, perp = _tc_stage(flat, embeddings)
    quant = _sc_gather(embeddings, idx2d.reshape(ROWS))
    return (
        quant.reshape(inputs.shape),
        idx2d.reshape(inputs.shape[:-1]),
        perp[0, 0],
    )


# speedup vs baseline: 1.2753x; 1.0158x over previous
import functools

import jax
import jax.numpy as jnp
from jax import lax
from jax.experimental import pallas as pl
from jax.experimental.pallas import tpu as pltpu
from jax.experimental.pallas import tpu_sc as plsc

E_DIM = 64
N_CODES = 1024
ROWS = 32 * 1024
BLK = 4096
N_BLK = ROWS // BLK


def _dist_argmin_body(x_ref, e_ref, idx_ref, perp_ref, counts_ref):
    i = pl.program_id(0)
    x = x_ref[...]
    e = e_ref[...]
    a_sq = jnp.sum(x * x, axis=1, keepdims=True)
    ab = lax.dot_general(x + x, e, (((1,), (1,)), ((), ())))
    b_sq = jnp.sum(e * e, axis=1)[None, :]
    dist = (a_sq - ab) + b_sq
    m = jnp.min(dist, axis=1, keepdims=True)
    eqm = dist == m
    lane_f = lax.broadcasted_iota(jnp.int32, (1, N_CODES), 1).astype(jnp.float32)
    idx_f = jnp.min(jnp.where(eqm, lane_f, float(N_CODES)), axis=1)
    idx_ref[...] = idx_f.astype(jnp.int32).reshape(BLK // 128, 128)
    cnt = jnp.sum(jnp.where(eqm, 1.0, 0.0), axis=0, keepdims=True)

    @pl.when(i == 0)
    def _():
        counts_ref[...] = cnt

    @pl.when(i > 0)
    def _():
        counts_ref[...] = counts_ref[...] + cnt

    @pl.when(i == N_BLK - 1)
    def _():
        p = counts_ref[...] * (1.0 / ROWS)
        ent = jnp.sum(p * jnp.log(p + 1e-10), axis=(0, 1), keepdims=True)
        perp_ref[...] = jnp.exp(-ent)


def _tc_stage(flat, emb):
    return pl.pallas_call(
        _dist_argmin_body,
        grid=(N_BLK,),
        in_specs=[
            pl.BlockSpec((BLK, E_DIM), lambda i: (i, 0)),
            pl.BlockSpec((N_CODES, E_DIM), lambda i: (0, 0)),
        ],
        out_specs=[
            pl.BlockSpec((BLK // 128, 128), lambda i: (i, 0)),
            pl.BlockSpec((1, 1), lambda i: (0, 0)),
        ],
        out_shape=[
            jax.ShapeDtypeStruct((ROWS // 128, 128), jnp.int32),
            jax.ShapeDtypeStruct((1, 1), jnp.float32),
        ],
        scratch_shapes=[pltpu.VMEM((1, N_CODES), jnp.float32)],
    )(flat, emb)


def _sc_gather(table, idx_flat):
    info = plsc.get_sparse_core_info()
    nc, ns = info.num_cores, info.num_subcores
    nw = nc * ns
    b_per_w = ROWS // nw
    mesh = plsc.VectorSubcoreMesh(core_axis_name="c", subcore_axis_name="s")

    @functools.partial(
        pl.kernel,
        mesh=mesh,
        compiler_params=pltpu.CompilerParams(use_tc_tiling_on_sc=False),
        out_type=jax.ShapeDtypeStruct((ROWS, E_DIM), jnp.float32),
        scratch_types=[
            pltpu.VMEM((b_per_w,), jnp.int32),
            pltpu.VMEM((b_per_w, E_DIM), jnp.float32),
            pltpu.SemaphoreType.DMA,
        ],
    )
    def gather_k(table_hbm, idx_hbm, out_hbm, idx_v, rows_v, sem):
        wid = lax.axis_index("s") * nc + lax.axis_index("c")
        base = wid * b_per_w
        pltpu.sync_copy(idx_hbm.at[pl.ds(base, b_per_w)], idx_v)
        pltpu.async_copy(table_hbm.at[idx_v], rows_v, sem).wait()
        pltpu.sync_copy(rows_v, out_hbm.at[pl.ds(base, b_per_w)])

    return gather_k(table, idx_flat)


def kernel(inputs, embeddings):
    flat = inputs.reshape(ROWS, E_DIM)
    idx2d, perp = _tc_stage(flat, embeddings)
    quant = _sc_gather(embeddings, idx2d.reshape(ROWS))
    return (
        quant.reshape(inputs.shape),
        idx2d.reshape(inputs.shape[:-1]),
        perp[0, 0],
    )
